# baseline (speedup 1.0000x reference)
import functools

import jax
import jax.numpy as jnp
from jax import lax
from jax.experimental import pallas as pl
from jax.experimental.pallas import tpu as pltpu
from jax.experimental.pallas import tpu_sc as plsc

_NC = 2
_NS = 16
_NW = _NC * _NS
_CH = 24


def _make_gather(vocab: int, d: int, b: int):
  b_per_w = b // _NW
  sizes = [_CH] * (b_per_w // _CH)
  if b_per_w % _CH:
    sizes.append(b_per_w % _CH)
  offs = [sum(sizes[:i]) for i in range(len(sizes))]
  nchunk = len(sizes)
  mesh = plsc.VectorSubcoreMesh(
      core_axis_name="c", subcore_axis_name="s",
      num_cores=_NC, num_subcores=_NS)

  @functools.partial(
      pl.kernel,
      out_type=jax.ShapeDtypeStruct((b, d), jnp.float32),
      mesh=mesh,
      scratch_types=[
          pltpu.VMEM((b_per_w,), jnp.int32),
          pltpu.VMEM((2, _CH, d), jnp.float32),
          pltpu.SemaphoreType.DMA,
          pltpu.SemaphoreType.DMA,
          pltpu.SemaphoreType.DMA,
          pltpu.SemaphoreType.DMA,
      ],
  )
  def gather(ids_hbm, table_hbm, out_hbm, idx_v, rows_v, g0, g1, o0, o1):
    gsem = (g0, g1)
    osem = (o0, o1)
    wid = lax.axis_index("s") * _NC + lax.axis_index("c")
    base = wid * b_per_w
    pltpu.sync_copy(ids_hbm.at[pl.ds(base, b_per_w)], idx_v)

    def gather_desc(c):
      buf = c % 2
      idx = idx_v.at[pl.ds(offs[c], sizes[c])]
      return pltpu.make_async_copy(
          table_hbm.at[idx], rows_v.at[buf, pl.ds(0, sizes[c])], gsem[buf])

    def out_desc(c):
      buf = c % 2
      return pltpu.make_async_copy(
          rows_v.at[buf, pl.ds(0, sizes[c])],
          out_hbm.at[pl.ds(base + offs[c], sizes[c])], osem[buf])

    gather_desc(0).start()
    gather_desc(1).start()
    for c in range(nchunk):
      gather_desc(c).wait()
      out_desc(c).start()
      if c + 2 < nchunk:
        out_desc(c).wait()
        gather_desc(c + 2).start()
    out_desc(nchunk - 2).wait()
    out_desc(nchunk - 1).wait()

  return gather


def kernel(input_ids, embed):
  bsz, seq = input_ids.shape
  vocab, d = embed.shape
  flat = input_ids.reshape(bsz * seq)
  out = _make_gather(vocab, d, bsz * seq)(flat, embed)
  return out.reshape(bsz, seq, d)


# speedup vs baseline: 1.0170x; 1.0170x over previous
import functools

import jax
import jax.numpy as jnp
from jax import lax
from jax.experimental import pallas as pl
from jax.experimental.pallas import tpu as pltpu
from jax.experimental.pallas import tpu_sc as plsc

_NC = 2
_NS = 16
_NW = _NC * _NS
_CH = 16


def _make_gather(vocab: int, d: int, b: int):
  b_per_w = b // _NW
  nchunk = b_per_w // _CH
  mesh = plsc.VectorSubcoreMesh(
      core_axis_name="c", subcore_axis_name="s",
      num_cores=_NC, num_subcores=_NS)

  @functools.partial(
      pl.kernel,
      out_type=jax.ShapeDtypeStruct((b, d), jnp.float32),
      mesh=mesh,
      scratch_types=[
          pltpu.VMEM((b_per_w,), jnp.int32),
          pltpu.VMEM((2, _CH, d), jnp.float32),
          pltpu.SemaphoreType.DMA,
          pltpu.SemaphoreType.DMA,
          pltpu.SemaphoreType.DMA,
          pltpu.SemaphoreType.DMA,
      ],
  )
  def gather(ids_hbm, table_hbm, out_hbm, idx_v, rows_v, g0, g1, o0, o1):
    gsem = (g0, g1)
    osem = (o0, o1)
    wid = lax.axis_index("s") * _NC + lax.axis_index("c")
    base = wid * b_per_w
    pltpu.sync_copy(ids_hbm.at[pl.ds(base, b_per_w)], idx_v)

    def gather_desc(c, buf):
      idx = idx_v.at[pl.ds(c * _CH, _CH)]
      return pltpu.make_async_copy(table_hbm.at[idx], rows_v.at[buf], gsem[buf])

    def out_desc(c, buf):
      return pltpu.make_async_copy(
          rows_v.at[buf], out_hbm.at[pl.ds(base + c * _CH, _CH)], osem[buf])

    gather_desc(0, 0).start()
    gather_desc(1, 1).start()

    def body(g, carry):
      for buf in (0, 1):
        c = 2 * g + buf
        gather_desc(c, buf).wait()
        out_desc(c, buf).start()
        out_desc(c, buf).wait()
        gather_desc(c + 2, buf).start()
      return carry

    lax.fori_loop(0, nchunk // 2 - 1, body, 0, unroll=False)
    for buf in (0, 1):
      c = nchunk - 2 + buf
      gather_desc(c, buf).wait()
      out_desc(c, buf).start()
    out_desc(nchunk - 2, 0).wait()
    out_desc(nchunk - 1, 1).wait()

  return gather


def kernel(input_ids, embed):
  bsz, seq = input_ids.shape
  vocab, d = embed.shape
  flat = input_ids.reshape(bsz * seq)
  out = _make_gather(vocab, d, bsz * seq)(flat, embed)
  return out.reshape(bsz, seq, d)


# speedup vs baseline: 1.0203x; 1.0033x over previous
import functools

import jax
import jax.numpy as jnp
from jax import lax
from jax.experimental import pallas as pl
from jax.experimental.pallas import tpu as pltpu
from jax.experimental.pallas import tpu_sc as plsc

_NC = 2
_NS = 16
_NW = _NC * _NS
_CH = 16


def _make_gather(vocab: int, d: int, b: int):
  b_per_w = b // _NW
  nchunk = b_per_w // _CH
  mesh = plsc.VectorSubcoreMesh(
      core_axis_name="c", subcore_axis_name="s",
      num_cores=_NC, num_subcores=_NS)

  @functools.partial(
      pl.kernel,
      out_type=jax.ShapeDtypeStruct((b, d), jnp.float32),
      mesh=mesh,
      scratch_types=[
          pltpu.VMEM((b_per_w,), jnp.int32),
          pltpu.VMEM((2, _CH, d), jnp.float32),
          pltpu.SemaphoreType.DMA,
          pltpu.SemaphoreType.DMA,
          pltpu.SemaphoreType.DMA,
          pltpu.SemaphoreType.DMA,
      ],
  )
  def gather(ids_hbm, table_hbm, out_hbm, idx_v, rows_v, g0, g1, o0, o1):
    gsem = (g0, g1)
    osem = (o0, o1)
    wid = lax.axis_index("s") * _NC + lax.axis_index("c")
    base = wid * b_per_w
    pltpu.sync_copy(ids_hbm.at[pl.ds(base, b_per_w)], idx_v)

    def gather_desc(c, buf):
      idx = idx_v.at[pl.ds(c * _CH, _CH)]
      return pltpu.make_async_copy(table_hbm.at[idx], rows_v.at[buf], gsem[buf])

    def out_desc(c, buf):
      return pltpu.make_async_copy(
          rows_v.at[buf], out_hbm.at[pl.ds(base + c * _CH, _CH)], osem[buf])

    gather_desc(0, 0).start()
    gather_desc(1, 1).start()

    def body(g, carry):
      for buf in (0, 1):
        c = 2 * g + buf
        gather_desc(c, buf).wait()
        out_desc(c, buf).start()

        @pl.when(c + 2 < nchunk)
        def _():
          out_desc(c, buf).wait()
          gather_desc(c + 2, buf).start()

      return carry

    lax.fori_loop(0, nchunk // 2, body, 0, unroll=False)
    out_desc(nchunk - 2, 0).wait()
    out_desc(nchunk - 1, 1).wait()

  return gather


def kernel(input_ids, embed):
  bsz, seq = input_ids.shape
  vocab, d = embed.shape
  flat = input_ids.reshape(bsz * seq)
  out = _make_gather(vocab, d, bsz * seq)(flat, embed)
  return out.reshape(bsz, seq, d)
